# baseline (speedup 1.0000x reference)
import functools

import jax
import jax.numpy as jnp
from jax import lax
from jax.experimental import pallas as pl
from jax.experimental.pallas import tpu as pltpu
from jax.experimental.pallas import tpu_sc as plsc

_CHANNELS = 128
_NUM_LAYERS = 12
_MIN_RATIO = 0.2
_N_NODES = 100000
_N_EDGES = 3200000
_B = 128

_HIST = 102400
_CH_ROWS = 8
_CHUNK = _CH_ROWS * 128
_N_WORKERS = 32
_CHUNKS_PER_W = 98
_PAIRS = _CHUNKS_PER_W // 2
_EPAD = _N_WORKERS * _CHUNKS_PER_W * _CHUNK

_NB = 20
_BLK = _N_NODES // _NB

_HP = lax.Precision.HIGHEST


def _deg_body(edges_hbm, zeros_hbm, ones_hbm, out_hbm, idx0, idx1, ones_v,
              hist_sh, gsem0, gsem1, ssem):
    c = lax.axis_index("c")
    s = lax.axis_index("s")
    wid = s * 2 + c
    base = wid * _CHUNKS_PER_W
    pltpu.sync_copy(zeros_hbm.at[pl.ds(s * 6400, 6400)],
                    hist_sh.at[pl.ds(s * 6400, 6400)])
    pltpu.sync_copy(ones_hbm, ones_v)
    plsc.subcore_barrier()
    pltpu.sync_copy(edges_hbm.at[base], idx0)

    def body(jj, carry):
        sc0 = [pltpu.async_copy(ones_v.at[r], hist_sh.at[idx0.at[r]], ssem,
                                add=True) for r in range(_CH_ROWS)]
        g1 = pltpu.async_copy(edges_hbm.at[base + 2 * jj + 1], idx1, gsem1)
        for d in sc0:
            d.wait()
        g1.wait()
        sc1 = [pltpu.async_copy(ones_v.at[r], hist_sh.at[idx1.at[r]], ssem,
                                add=True) for r in range(_CH_ROWS)]

        @pl.when(jj < _PAIRS - 1)
        def _():
            pltpu.async_copy(edges_hbm.at[base + 2 * jj + 2], idx0, gsem0)

        for d in sc1:
            d.wait()

        @pl.when(jj < _PAIRS - 1)
        def _():
            pltpu.make_async_copy(edges_hbm.at[base], idx0, gsem0).wait()

        return carry

    lax.fori_loop(0, _PAIRS, body, 0)
    plsc.subcore_barrier()
    pltpu.sync_copy(hist_sh.at[pl.ds(s * 6400, 6400)],
                    out_hbm.at[c, pl.ds(s * 6400, 6400)])


def _deg_kernel(edges3, zeros, ones):
    return functools.partial(
        pl.kernel,
        out_type=jax.ShapeDtypeStruct((2, _HIST), jnp.float32),
        mesh=plsc.VectorSubcoreMesh(core_axis_name="c", subcore_axis_name="s"),
        scratch_types=[
            pltpu.VMEM((_CH_ROWS, 128), jnp.int32),
            pltpu.VMEM((_CH_ROWS, 128), jnp.int32),
            pltpu.VMEM((_CH_ROWS, 128), jnp.float32),
            pltpu.VMEM_SHARED((_HIST,), jnp.float32),
            pltpu.SemaphoreType.DMA,
            pltpu.SemaphoreType.DMA,
            pltpu.SemaphoreType.DMA,
        ],
    )(_deg_body)(edges3, zeros, ones)


def _gsum_body(deg4_hbm, batchg_hbm, zeros_hbm, out_hbm, d0v, d1v, dv, sv,
               gi, accd_sh, accs_sh):
    c = lax.axis_index("c")
    s = lax.axis_index("s")
    wid = s * 2 + c

    @pl.when(s == 0)
    def _():
        pltpu.sync_copy(zeros_hbm.at[pl.ds(0, 256)], accd_sh)
        pltpu.sync_copy(zeros_hbm.at[pl.ds(0, 256)], accs_sh)

    pltpu.sync_copy(deg4_hbm.at[0, wid], d0v)
    pltpu.sync_copy(deg4_hbm.at[1, wid], d1v)
    pltpu.sync_copy(batchg_hbm.at[wid], gi)

    def compute(r, carry):
        for k in range(8):
            sl = pl.ds(k * 16, 16)
            v = d0v[r, sl] + d1v[r, sl]
            dv[r, sl] = v
            sv[r, sl] = v * v
        return carry

    lax.fori_loop(0, 25, compute, 0)
    plsc.subcore_barrier()

    def scatter(r, carry):
        pltpu.sync_copy(dv.at[r], accd_sh.at[gi.at[r]], add=True)
        pltpu.sync_copy(sv.at[r], accs_sh.at[gi.at[r]], add=True)
        return carry

    lax.fori_loop(0, 25, scatter, 0)
    plsc.subcore_barrier()

    @pl.when(s == 0)
    def _():
        pltpu.sync_copy(accd_sh.at[pl.ds(0, 128)], out_hbm.at[2 * c])
        pltpu.sync_copy(accs_sh.at[pl.ds(0, 128)], out_hbm.at[2 * c + 1])


def _gsum_kernel(deg4, batchg, zeros):
    return functools.partial(
        pl.kernel,
        out_type=jax.ShapeDtypeStruct((4, 128), jnp.float32),
        mesh=plsc.VectorSubcoreMesh(core_axis_name="c", subcore_axis_name="s"),
        scratch_types=[
            pltpu.VMEM((25, 128), jnp.float32),
            pltpu.VMEM((25, 128), jnp.float32),
            pltpu.VMEM((25, 128), jnp.float32),
            pltpu.VMEM((25, 128), jnp.float32),
            pltpu.VMEM((25, 128), jnp.int32),
            pltpu.VMEM_SHARED((256,), jnp.float32),
            pltpu.VMEM_SHARED((256,), jnp.float32),
        ],
    )(_gsum_body)(deg4, batchg, zeros)


def _mask_t(batch_row):
    return (lax.broadcasted_iota(jnp.int32, (_B, _BLK), 0) == batch_row
            ).astype(jnp.float32)


def _pool_body(batch_ref, emb_ref, acc_ref):
    i = pl.program_id(0)

    @pl.when(i == 0)
    def _():
        acc_ref[...] = jnp.zeros_like(acc_ref)

    mt = _mask_t(batch_ref[0, 0, :][None, :])
    pooled = lax.dot_general(mt, emb_ref[...], (((1,), (0,)), ((), ())),
                             preferred_element_type=jnp.float32,
                             precision=_HP)
    cnt = lax.dot_general(mt, jnp.ones((_BLK, 8), jnp.float32),
                          (((1,), (0,)), ((), ())),
                          preferred_element_type=jnp.float32, precision=_HP)
    acc_ref[...] += jnp.concatenate([pooled, cnt], axis=1)


def _pool_call(batch3, emb, lo, n):
    return pl.pallas_call(
        _pool_body,
        grid=(n,),
        in_specs=[
            pl.BlockSpec((1, 1, _BLK), lambda i: (i + lo, 0, 0)),
            pl.BlockSpec((_BLK, _CHANNELS), lambda i: (i + lo, 0)),
        ],
        out_specs=pl.BlockSpec((_B, _CHANNELS + 8), lambda i: (0, 0)),
        out_shape=jax.ShapeDtypeStruct((_B, _CHANNELS + 8), jnp.float32),
    )(batch3, emb)


def _head_body(gsum_ref, pool1_ref, pool2_ref, W1_ref, b1_ref, W2_ref,
               b2_ref, Wt_ref, bt_ref, Wl_ref, bl_ref, tr_ref, lg_ref):
    sdeg = gsum_ref[0, :] + gsum_ref[2, :]
    sdeg2 = gsum_ref[1, :] + gsum_ref[3, :]
    pool = pool1_ref[...] + pool2_ref[...]
    pooled_sum = pool[:, :_CHANNELS]
    n = pool[:, _CHANNELS]
    counts = jnp.maximum(n, 1.0)
    log_n = jnp.log(n + 1.0)
    log_e = jnp.log(0.5 * sdeg + 1.0)
    density = sdeg / (n * (n - 1.0) + 1e-08)
    avg_deg = sdeg / counts
    deg_var = jnp.clip(sdeg2 / counts - avg_deg * avg_deg, 0.0, None)
    pooled = pooled_sum / counts[:, None]
    feats = jnp.concatenate(
        [log_n[:, None], log_e[:, None], density[:, None],
         avg_deg[:, None], deg_var[:, None], pooled], axis=1)
    mm = functools.partial(jnp.matmul, precision=_HP)
    h = jnp.maximum(mm(feats, W1_ref[...]) + b1_ref[...], 0.0)
    h = jnp.maximum(mm(h, W2_ref[...]) + b2_ref[...], 0.0)
    sig_t = 1.0 / (1.0 + jnp.exp(-(mm(h, Wt_ref[...]) + bt_ref[...])))
    sig_l = 1.0 / (1.0 + jnp.exp(-(mm(h, Wl_ref[...]) + bl_ref[...])))
    tr_ref[...] = _MIN_RATIO + (1.0 - _MIN_RATIO) * sig_t
    lg_ref[...] = sig_l


def _full(shape):
    return pl.BlockSpec(shape, lambda: tuple(0 for _ in shape))


def _head_call(gsum, pool1, pool2, W1, b1, W2, b2, Wt, bt, Wl, bl):
    return pl.pallas_call(
        _head_body,
        in_specs=[
            _full(gsum.shape),
            _full(pool1.shape),
            _full(pool2.shape),
            _full(W1.shape), _full(b1.shape),
            _full(W2.shape), _full(b2.shape),
            _full(Wt.shape), _full(bt.shape),
            _full(Wl.shape), _full(bl.shape),
        ],
        out_specs=[
            _full((_B, _NUM_LAYERS)),
            _full((_B, _NUM_LAYERS)),
        ],
        out_shape=[
            jax.ShapeDtypeStruct((_B, _NUM_LAYERS), jnp.float32),
            jax.ShapeDtypeStruct((_B, _NUM_LAYERS), jnp.float32),
        ],
    )(gsum, pool1, pool2, W1, b1, W2, b2, Wt, bt, Wl, bl)


def kernel(x, edge_index, batch, node_emb, W1, b1, W2, b2, Wt, bt, Wl, bl):
    src = edge_index[0].astype(jnp.int32)
    src = jnp.concatenate(
        [src, jnp.full((_EPAD - _N_EDGES,), _N_NODES, jnp.int32)])
    edges3 = src.reshape(_N_WORKERS * _CHUNKS_PER_W, _CH_ROWS, 128)
    zeros = jnp.zeros((_HIST,), jnp.float32)
    ones = jnp.ones((_CH_ROWS, 128), jnp.float32)

    deg2 = _deg_kernel(edges3, zeros, ones)

    bi = batch.astype(jnp.int32)
    batch3 = bi.reshape(_NB, 1, _BLK)
    pool1 = _pool_call(batch3, node_emb, 0, 15)

    batchg = jnp.concatenate(
        [bi, jnp.full((_HIST - _N_NODES,), 255, jnp.int32)]
    ).reshape(_N_WORKERS, 25, 128)
    deg4 = deg2.reshape(2, _N_WORKERS, 25, 128)
    gsum = _gsum_kernel(deg4, batchg, zeros)

    dep = (deg2[0, 0] * 0.0).astype(jnp.int32)
    batch3b = batch3 + dep
    pool2 = _pool_call(batch3b, node_emb, 15, 5)

    token_ratios, layer_gates = _head_call(
        gsum, pool1, pool2,
        W1, b1.reshape(1, -1), W2, b2.reshape(1, -1),
        Wt, bt.reshape(1, -1), Wl, bl.reshape(1, -1))
    return (token_ratios, layer_gates)


# speedup vs baseline: 1.4159x; 1.4159x over previous
import functools

import jax
import jax.numpy as jnp
from jax import lax
from jax.experimental import pallas as pl
from jax.experimental.pallas import tpu as pltpu
from jax.experimental.pallas import tpu_sc as plsc

_CHANNELS = 128
_NUM_LAYERS = 12
_MIN_RATIO = 0.2
_N_NODES = 100000
_N_EDGES = 3200000
_B = 128

_HIST = 102400
_CH_ROWS = 8
_CHUNK = _CH_ROWS * 128
_N_WORKERS = 32
_CHUNKS_PER_W = 98
_PAIRS = _CHUNKS_PER_W // 2
_EPAD = _N_WORKERS * _CHUNKS_PER_W * _CHUNK

_NB = 20
_BLK = _N_NODES // _NB

_HP = lax.Precision.HIGHEST


def _deg_body(edges_hbm, zeros_hbm, ones_hbm, out_hbm, idx0, idx1, ones_v,
              hist_sh, gsem0, gsem1, ssem):
    c = lax.axis_index("c")
    s = lax.axis_index("s")
    wid = s * 2 + c
    base = wid * _CHUNKS_PER_W
    pltpu.sync_copy(zeros_hbm.at[pl.ds(s * 6400, 6400)],
                    hist_sh.at[pl.ds(s * 6400, 6400)])
    pltpu.sync_copy(ones_hbm, ones_v)
    plsc.subcore_barrier()
    pltpu.sync_copy(edges_hbm.at[base], idx0)

    def body(jj, carry):
        sc0 = [pltpu.async_copy(ones_v.at[r], hist_sh.at[idx0.at[r]], ssem,
                                add=True) for r in range(_CH_ROWS)]
        g1 = pltpu.async_copy(edges_hbm.at[base + 2 * jj + 1], idx1, gsem1)
        for d in sc0:
            d.wait()
        g1.wait()
        sc1 = [pltpu.async_copy(ones_v.at[r], hist_sh.at[idx1.at[r]], ssem,
                                add=True) for r in range(_CH_ROWS)]

        @pl.when(jj < _PAIRS - 1)
        def _():
            pltpu.async_copy(edges_hbm.at[base + 2 * jj + 2], idx0, gsem0)

        for d in sc1:
            d.wait()

        @pl.when(jj < _PAIRS - 1)
        def _():
            pltpu.make_async_copy(edges_hbm.at[base], idx0, gsem0).wait()

        return carry

    lax.fori_loop(0, _PAIRS, body, 0)
    plsc.subcore_barrier()
    pltpu.sync_copy(hist_sh.at[pl.ds(s * 6400, 6400)],
                    out_hbm.at[c, pl.ds(s * 6400, 6400)])


def _deg_kernel(edges3, zeros, ones):
    return functools.partial(
        pl.kernel,
        out_type=jax.ShapeDtypeStruct((2, _HIST), jnp.float32),
        mesh=plsc.VectorSubcoreMesh(core_axis_name="c", subcore_axis_name="s"),
        scratch_types=[
            pltpu.VMEM((_CH_ROWS, 128), jnp.int32),
            pltpu.VMEM((_CH_ROWS, 128), jnp.int32),
            pltpu.VMEM((_CH_ROWS, 128), jnp.float32),
            pltpu.VMEM_SHARED((_HIST,), jnp.float32),
            pltpu.SemaphoreType.DMA,
            pltpu.SemaphoreType.DMA,
            pltpu.SemaphoreType.DMA,
        ],
    )(_deg_body)(edges3, zeros, ones)


def _gsum_body(deg4_hbm, batchg_hbm, zeros_hbm, out_hbm, d0v, d1v, dv, sv,
               gi, accd_sh, accs_sh):
    c = lax.axis_index("c")
    s = lax.axis_index("s")
    wid = s * 2 + c

    @pl.when(s == 0)
    def _():
        pltpu.sync_copy(zeros_hbm.at[pl.ds(0, 256)], accd_sh)
        pltpu.sync_copy(zeros_hbm.at[pl.ds(0, 256)], accs_sh)

    pltpu.sync_copy(deg4_hbm.at[0, wid], d0v)
    pltpu.sync_copy(deg4_hbm.at[1, wid], d1v)
    pltpu.sync_copy(batchg_hbm.at[wid], gi)

    def compute(r, carry):
        for k in range(8):
            sl = pl.ds(k * 16, 16)
            v = d0v[r, sl] + d1v[r, sl]
            dv[r, sl] = v
            sv[r, sl] = v * v
        return carry

    lax.fori_loop(0, 25, compute, 0)
    plsc.subcore_barrier()

    def scatter(r, carry):
        pltpu.sync_copy(dv.at[r], accd_sh.at[gi.at[r]], add=True)
        pltpu.sync_copy(sv.at[r], accs_sh.at[gi.at[r]], add=True)
        return carry

    lax.fori_loop(0, 25, scatter, 0)
    plsc.subcore_barrier()

    @pl.when(s == 0)
    def _():
        pltpu.sync_copy(accd_sh.at[pl.ds(0, 128)], out_hbm.at[2 * c])
        pltpu.sync_copy(accs_sh.at[pl.ds(0, 128)], out_hbm.at[2 * c + 1])


def _gsum_kernel(deg4, batchg, zeros):
    return functools.partial(
        pl.kernel,
        out_type=jax.ShapeDtypeStruct((4, 128), jnp.float32),
        mesh=plsc.VectorSubcoreMesh(core_axis_name="c", subcore_axis_name="s"),
        scratch_types=[
            pltpu.VMEM((25, 128), jnp.float32),
            pltpu.VMEM((25, 128), jnp.float32),
            pltpu.VMEM((25, 128), jnp.float32),
            pltpu.VMEM((25, 128), jnp.float32),
            pltpu.VMEM((25, 128), jnp.int32),
            pltpu.VMEM_SHARED((256,), jnp.float32),
            pltpu.VMEM_SHARED((256,), jnp.float32),
        ],
    )(_gsum_body)(deg4, batchg, zeros)


def _mask_t(batch_row):
    return (lax.broadcasted_iota(jnp.int32, (_B, _BLK), 0) == batch_row
            ).astype(jnp.float32)


def _pool_body(batch_ref, emb_ref, acc_ref):
    i = pl.program_id(0)

    @pl.when(i == 0)
    def _():
        acc_ref[...] = jnp.zeros_like(acc_ref)

    mt = _mask_t(batch_ref[0, 0, :][None, :])
    pooled = lax.dot_general(mt, emb_ref[...], (((1,), (0,)), ((), ())),
                             preferred_element_type=jnp.float32,
                             precision=_HP)
    cnt = lax.dot_general(mt, jnp.ones((_BLK, 8), jnp.float32),
                          (((1,), (0,)), ((), ())),
                          preferred_element_type=jnp.float32, precision=_HP)
    acc_ref[...] += jnp.concatenate([pooled, cnt], axis=1)


def _pool_call(batch3, emb, lo, n):
    return pl.pallas_call(
        _pool_body,
        grid=(n,),
        in_specs=[
            pl.BlockSpec((1, 1, _BLK), lambda i: (i + lo, 0, 0)),
            pl.BlockSpec((_BLK, _CHANNELS), lambda i: (i + lo, 0)),
        ],
        out_specs=pl.BlockSpec((_B, _CHANNELS + 8), lambda i: (0, 0)),
        out_shape=jax.ShapeDtypeStruct((_B, _CHANNELS + 8), jnp.float32),
    )(batch3, emb)


def _head_body(gsum_ref, pool1_ref, pool2_ref, W1_ref, b1_ref, W2_ref,
               b2_ref, Wt_ref, bt_ref, Wl_ref, bl_ref, tr_ref, lg_ref):
    sdeg = gsum_ref[0, :] + gsum_ref[2, :]
    sdeg2 = gsum_ref[1, :] + gsum_ref[3, :]
    pool = pool1_ref[...] + pool2_ref[...]
    pooled_sum = pool[:, :_CHANNELS]
    n = pool[:, _CHANNELS]
    counts = jnp.maximum(n, 1.0)
    log_n = jnp.log(n + 1.0)
    log_e = jnp.log(0.5 * sdeg + 1.0)
    density = sdeg / (n * (n - 1.0) + 1e-08)
    avg_deg = sdeg / counts
    deg_var = jnp.clip(sdeg2 / counts - avg_deg * avg_deg, 0.0, None)
    pooled = pooled_sum / counts[:, None]
    feats = jnp.concatenate(
        [log_n[:, None], log_e[:, None], density[:, None],
         avg_deg[:, None], deg_var[:, None], pooled], axis=1)
    mm = functools.partial(jnp.matmul, precision=_HP)
    h = jnp.maximum(mm(feats, W1_ref[...]) + b1_ref[...], 0.0)
    h = jnp.maximum(mm(h, W2_ref[...]) + b2_ref[...], 0.0)
    sig_t = 1.0 / (1.0 + jnp.exp(-(mm(h, Wt_ref[...]) + bt_ref[...])))
    sig_l = 1.0 / (1.0 + jnp.exp(-(mm(h, Wl_ref[...]) + bl_ref[...])))
    tr_ref[...] = _MIN_RATIO + (1.0 - _MIN_RATIO) * sig_t
    lg_ref[...] = sig_l


def _full(shape):
    return pl.BlockSpec(shape, lambda: tuple(0 for _ in shape))


def _head_call(gsum, pool1, pool2, W1, b1, W2, b2, Wt, bt, Wl, bl):
    return pl.pallas_call(
        _head_body,
        in_specs=[
            _full(gsum.shape),
            _full(pool1.shape),
            _full(pool2.shape),
            _full(W1.shape), _full(b1.shape),
            _full(W2.shape), _full(b2.shape),
            _full(Wt.shape), _full(bt.shape),
            _full(Wl.shape), _full(bl.shape),
        ],
        out_specs=[
            _full((_B, _NUM_LAYERS)),
            _full((_B, _NUM_LAYERS)),
        ],
        out_shape=[
            jax.ShapeDtypeStruct((_B, _NUM_LAYERS), jnp.float32),
            jax.ShapeDtypeStruct((_B, _NUM_LAYERS), jnp.float32),
        ],
    )(gsum, pool1, pool2, W1, b1, W2, b2, Wt, bt, Wl, bl)


def kernel(x, edge_index, batch, node_emb, W1, b1, W2, b2, Wt, bt, Wl, bl):
    src = edge_index[0].astype(jnp.int32)
    src = jnp.concatenate(
        [src, jnp.full((_EPAD - _N_EDGES,), _N_NODES, jnp.int32)])
    edges3 = src.reshape(_N_WORKERS * _CHUNKS_PER_W, _CH_ROWS, 128)
    zeros = jnp.zeros((_HIST,), jnp.float32)
    ones = jnp.ones((_CH_ROWS, 128), jnp.float32)

    deg2 = _deg_kernel(edges3, zeros, ones)

    bi = batch.astype(jnp.int32)
    batch3 = bi.reshape(_NB, 1, _BLK)
    pool1 = _pool_call(batch3, node_emb, 0, 15)

    batchg = jnp.concatenate(
        [bi, jnp.full((_HIST - _N_NODES,), 255, jnp.int32)]
    ).reshape(_N_WORKERS, 25, 128)
    depp = (pool1[0, 0] * 0.0).astype(jnp.int32)
    deg4 = deg2.reshape(2, _N_WORKERS, 25, 128)
    gsum = _gsum_kernel(deg4, batchg + depp, zeros)

    dep = (deg2[0, 0] * 0.0).astype(jnp.int32)
    batch3b = batch3 + dep
    pool2 = _pool_call(batch3b, node_emb, 15, 5)

    token_ratios, layer_gates = _head_call(
        gsum, pool1, pool2,
        W1, b1.reshape(1, -1), W2, b2.reshape(1, -1),
        Wt, bt.reshape(1, -1), Wl, bl.reshape(1, -1))
    return (token_ratios, layer_gates)
